# baseline (speedup 1.0000x reference)
import functools

import jax
import jax.numpy as jnp
from jax import lax
from jax.experimental import pallas as pl
from jax.experimental.pallas import tpu as pltpu
from jax.experimental.pallas import tpu_sc as plsc

_ROWS = 65536
_BATCH = 256
_FLIP = 1024
_NW = 32
_PER_W = _ROWS // _NW
_CH = 64
_NBUF = 6
_LOOK = 3
_NCHUNK = _PER_W // _CH


def _sc_body(x_hbm, o_hbm, *rest):
    bufs = rest[:_NBUF]
    lsems = rest[_NBUF:2 * _NBUF]
    ssems = rest[2 * _NBUF:3 * _NBUF]
    wid = lax.axis_index("s") * 2 + lax.axis_index("c")
    base = wid * _PER_W

    def load(i, b):
        src = pl.multiple_of((base + i * _CH) ^ _FLIP, _CH)
        return pltpu.make_async_copy(
            x_hbm.at[pl.ds(src, _CH)], bufs[b], lsems[b])

    def store(i, b):
        dst = pl.multiple_of(base + i * _CH, _CH)
        return pltpu.make_async_copy(
            bufs[b], o_hbm.at[pl.ds(dst, _CH)], ssems[b])

    for i in range(_LOOK):
        load(i, i % _NBUF).start()
    for i in range(_NCHUNK):
        b = i % _NBUF
        nxt = i + _LOOK
        if nxt < _NCHUNK:
            bn = nxt % _NBUF
            if nxt >= _NBUF:
                store(nxt - _NBUF, bn).wait()
            load(nxt, bn).start()
        load(i, b).wait()
        store(i, b).start()
    for i in range(max(0, _NCHUNK - _NBUF), _NCHUNK):
        store(i, i % _NBUF).wait()


@functools.partial(jax.jit, donate_argnums=())
def _sc_swap(x):
    mesh = plsc.VectorSubcoreMesh(core_axis_name="c", subcore_axis_name="s")
    scratch = [pltpu.VMEM((_CH, _BATCH), jnp.float32) for _ in range(_NBUF)]
    scratch += [pltpu.SemaphoreType.DMA for _ in range(2 * _NBUF)]
    return pl.kernel(
        _sc_body,
        mesh=mesh,
        out_type=jax.ShapeDtypeStruct((_ROWS, _BATCH), jnp.float32),
        scratch_types=scratch,
    )(x)


def kernel(x):
    return _sc_swap(x)


# speedup vs baseline: 1.5523x; 1.5523x over previous
import functools

import jax
import jax.numpy as jnp
from jax.experimental import pallas as pl
from jax.experimental.pallas import tpu as pltpu

_ROWS = 65536
_BATCH = 256
_BR = 1024
_NB = _ROWS // _BR
_NBUF = 8
_LOOK = 4


def _body(x_ref, o_ref, *rest):
    bufs = rest[:_NBUF]
    lsems = rest[_NBUF:2 * _NBUF]
    ssems = rest[2 * _NBUF:3 * _NBUF]

    def load(i, b):
        return pltpu.make_async_copy(
            x_ref.at[pl.ds((i ^ 1) * _BR, _BR)], bufs[b], lsems[b])

    def store(i, b):
        return pltpu.make_async_copy(
            bufs[b], o_ref.at[pl.ds(i * _BR, _BR)], ssems[b])

    for i in range(_LOOK):
        load(i, i % _NBUF).start()
    for i in range(_NB):
        b = i % _NBUF
        nxt = i + _LOOK
        if nxt < _NB:
            bn = nxt % _NBUF
            if nxt >= _NBUF:
                store(nxt - _NBUF, bn).wait()
            load(nxt, bn).start()
        load(i, b).wait()
        store(i, b).start()
    for i in range(_NB - _NBUF, _NB):
        store(i, i % _NBUF).wait()


@functools.partial(jax.jit, donate_argnums=())
def _tc_ring_swap(x):
    scratch = [pltpu.VMEM((_BR, _BATCH), jnp.float32) for _ in range(_NBUF)]
    scratch += [pltpu.SemaphoreType.DMA for _ in range(2 * _NBUF)]
    return pl.pallas_call(
        _body,
        in_specs=[pl.BlockSpec(memory_space=pl.ANY)],
        out_specs=pl.BlockSpec(memory_space=pl.ANY),
        out_shape=jax.ShapeDtypeStruct((_ROWS, _BATCH), jnp.float32),
        scratch_shapes=scratch,
    )(x)


def kernel(x):
    return _tc_ring_swap(x)


# speedup vs baseline: 1.5704x; 1.0117x over previous
import functools

import jax
import jax.numpy as jnp
from jax.experimental import pallas as pl
from jax.experimental.pallas import tpu as pltpu

_ROWS = 65536
_BATCH = 256
_BR = 1024
_NB = _ROWS // _BR
_NBUF = 12
_LOOK = 6


def _body(x_ref, o_ref, *rest):
    bufs = rest[:_NBUF]
    lsems = rest[_NBUF:2 * _NBUF]
    ssems = rest[2 * _NBUF:3 * _NBUF]

    def load(i, b):
        return pltpu.make_async_copy(
            x_ref.at[pl.ds((i ^ 1) * _BR, _BR)], bufs[b], lsems[b])

    def store(i, b):
        return pltpu.make_async_copy(
            bufs[b], o_ref.at[pl.ds(i * _BR, _BR)], ssems[b])

    for i in range(_LOOK):
        load(i, i % _NBUF).start()
    for i in range(_NB):
        b = i % _NBUF
        nxt = i + _LOOK
        if nxt < _NB:
            bn = nxt % _NBUF
            if nxt >= _NBUF:
                store(nxt - _NBUF, bn).wait()
            load(nxt, bn).start()
        load(i, b).wait()
        store(i, b).start()
    for i in range(_NB - _NBUF, _NB):
        store(i, i % _NBUF).wait()


@functools.partial(jax.jit, donate_argnums=())
def _tc_ring_swap(x):
    scratch = [pltpu.VMEM((_BR, _BATCH), jnp.float32) for _ in range(_NBUF)]
    scratch += [pltpu.SemaphoreType.DMA for _ in range(2 * _NBUF)]
    return pl.pallas_call(
        _body,
        in_specs=[pl.BlockSpec(memory_space=pl.ANY)],
        out_specs=pl.BlockSpec(memory_space=pl.ANY),
        out_shape=jax.ShapeDtypeStruct((_ROWS, _BATCH), jnp.float32),
        scratch_shapes=scratch,
    )(x)


def kernel(x):
    return _tc_ring_swap(x)
